# Initial kernel scaffold; baseline (speedup 1.0000x reference)
import jax
import jax.numpy as jnp
from jax.experimental import pallas as pl


def kernel(x):
    raise NotImplementedError("write your pallas kernel here")


# speedup vs baseline: 16.0637x; 16.0637x over previous
import functools

import numpy as np
import jax
import jax.numpy as jnp
from jax import lax
from jax.experimental import pallas as pl
from jax.experimental.pallas import tpu as pltpu
from jax.experimental.pallas import tpu_sc as plsc

_P = 0.5
_B, _S, _D = 4, 4096, 1024
_NC, _NS = 2, 16
_NW = _NC * _NS
_ROWS = _B * _S
_RPW = _ROWS // _NW
_CHUNK = 32
_NCHUNK = _RPW // _CHUNK
_NBUF = 3


def _src_rows() -> np.ndarray:
    rng = np.random.default_rng(0)
    mask = rng.random(_S - 1) < _P
    idxs = np.arange(_S - 1)[mask]
    perm = np.arange(_S)
    perm[idxs] = idxs + 1
    perm[idxs + 1] = idxs
    rows = np.arange(_B)[:, None] * _S + perm[None, :]
    return rows.astype(np.int32).reshape(_NW, _NCHUNK, _CHUNK)


_IDX = _src_rows()

_mesh = plsc.VectorSubcoreMesh(core_axis_name="c", subcore_axis_name="s")


@functools.partial(
    pl.kernel,
    mesh=_mesh,
    out_type=jax.ShapeDtypeStruct((_ROWS, _D), jnp.float32),
    scratch_types=[pltpu.VMEM((_NCHUNK, _CHUNK), jnp.int32)]
    + [pltpu.VMEM((_CHUNK, _D), jnp.float32) for _ in range(_NBUF)]
    + [pltpu.SemaphoreType.DMA for _ in range(2 * _NBUF)],
)
def _gather_rows(x_hbm, idx_hbm, out_hbm, idx_v, b0, b1, b2, g0, g1, g2,
                 w0, w1, w2):
    bufs = (b0, b1, b2)
    gsem = (g0, g1, g2)
    wsem = (w0, w1, w2)
    wid = lax.axis_index("s") * _NC + lax.axis_index("c")
    base = wid * _RPW

    pltpu.sync_copy(idx_hbm.at[wid], idx_v)

    gh = [None] * _NBUF
    for ci in range(_NBUF):
        gh[ci] = pltpu.async_copy(x_hbm.at[idx_v.at[ci]], bufs[ci], gsem[ci])
    for ci in range(_NCHUNK):
        s = ci % _NBUF
        gh[s].wait()
        w = pltpu.async_copy(
            bufs[s], out_hbm.at[pl.ds(base + ci * _CHUNK, _CHUNK)], wsem[s])
        w.wait()
        nx = ci + _NBUF
        if nx < _NCHUNK:
            gh[s] = pltpu.async_copy(
                x_hbm.at[idx_v.at[nx]], bufs[s], gsem[s])


@jax.jit
def kernel(x):
    out = _gather_rows(x.reshape(_ROWS, _D), jnp.asarray(_IDX))
    return out.reshape(_B, _S, _D)
